# baseline (speedup 1.0000x reference)
import functools

import jax
import jax.numpy as jnp
import numpy as _np
from jax import lax
from jax.experimental import pallas as pl
from jax.experimental.pallas import tpu as pltpu
from jax.experimental.pallas import tpu_sc as plsc

NC = 2
NS = 16
NW = NC * NS
LANES = 16

def _interleave_perm(f):
    perm = _np.empty((f,), dtype=_np.int32)
    for g in range(f // 32):
        for l in range(16):
            perm[32 * g + 2 * l] = 32 * g + l
            perm[32 * g + 2 * l + 1] = 32 * g + 16 + l
    return perm


def _ssp(v):
    return jnp.maximum(v, 0.0) + jnp.log1p(jnp.exp(-jnp.abs(v))) - 0.6931471805599453


def _h_body(x_ref, w1t_ref, b1_ref, o_ref):
    o_ref[...] = jnp.dot(x_ref[...], w1t_ref[...],
                         preferred_element_type=jnp.float32) + b1_ref[...]


def _compute_h(x, W1, b1):
    n, d = x.shape
    blk = 1000
    grid = n // blk
    return pl.pallas_call(
        _h_body,
        grid=(grid,),
        in_specs=[
            pl.BlockSpec((blk, d), lambda i: (i, 0)),
            pl.BlockSpec((d, W1.shape[0]), lambda i: (0, 0)),
            pl.BlockSpec((1, W1.shape[0]), lambda i: (0, 0)),
        ],
        out_specs=pl.BlockSpec((blk, W1.shape[0]), lambda i: (i, 0)),
        out_shape=jax.ShapeDtypeStruct((n, W1.shape[0]), jnp.float32),
    )(x, W1.T, b1.reshape(1, -1))


def _pack_bf16_pair(v):
    f = v.shape[1]
    a = lax.bitcast_convert_type(v[:, :f // 2].astype(jnp.bfloat16),
                                 jnp.uint16).astype(jnp.uint32)
    b = lax.bitcast_convert_type(v[:, f // 2:].astype(jnp.bfloat16),
                                 jnp.uint16).astype(jnp.uint32)
    return lax.bitcast_convert_type(a | (b << 16), jnp.int32)


def _wij_body(f1_ref, f2_ref, wft_ref, bf_ref, rc1_ref, rc2_ref, o_ref):
    v1 = _ssp(jnp.dot(f1_ref[...], wft_ref[...],
                      preferred_element_type=jnp.float32) + bf_ref[...])
    v2 = _ssp(jnp.dot(f2_ref[...], wft_ref[...],
                      preferred_element_type=jnp.float32) + bf_ref[...])
    o_ref[...] = jnp.concatenate(
        [_pack_bf16_pair(v1 * rc1_ref[...]), _pack_bf16_pair(v2 * rc2_ref[...])],
        axis=1)


def _compute_wij(f_ij, Wf, bf, rcut):
    p, r = f_ij.shape
    f = Wf.shape[0]
    perm = _interleave_perm(f)
    perm2 = _np.concatenate([perm[0::2], perm[1::2]])
    blk = 2000
    grid = (p // 2) // blk
    return pl.pallas_call(
        _wij_body,
        grid=(grid,),
        in_specs=[
            pl.BlockSpec((blk, r), lambda i: (i, 0)),
            pl.BlockSpec((blk, r), lambda i: (i + grid, 0)),
            pl.BlockSpec((r, f), lambda i: (0, 0)),
            pl.BlockSpec((1, f), lambda i: (0, 0)),
            pl.BlockSpec((blk, 1), lambda i: (i, 0)),
            pl.BlockSpec((blk, 1), lambda i: (i + grid, 0)),
        ],
        out_specs=pl.BlockSpec((blk, f), lambda i: (i, 0)),
        out_shape=jax.ShapeDtypeStruct((p // 2, f), jnp.int32),
    )(f_ij, f_ij, Wf.T[:, perm2], bf[perm2].reshape(1, -1),
      rcut.reshape(-1, 1), rcut.reshape(-1, 1))


def _out_body(p_ref, w2t_ref, b2_ref, o_ref):
    acc = p_ref[0] + p_ref[1]
    o_ref[...] = _ssp(jnp.dot(acc, w2t_ref[...],
                              preferred_element_type=jnp.float32) + b2_ref[...])


def _compute_out(parts, W2, b2):
    _, n, f = parts.shape
    d = W2.shape[0]
    blk = 1000
    grid = n // blk
    return pl.pallas_call(
        _out_body,
        grid=(grid,),
        in_specs=[
            pl.BlockSpec((2, blk, f), lambda i: (0, i, 0)),
            pl.BlockSpec((f, d), lambda i: (0, 0)),
            pl.BlockSpec((1, d), lambda i: (0, 0)),
        ],
        out_specs=pl.BlockSpec((blk, d), lambda i: (i, 0)),
        out_shape=jax.ShapeDtypeStruct((n, d), jnp.float32),
    )(parts, W2.T, b2.reshape(1, -1))


def _make_sc_scatter(n, d, p, chunk):
    nz_tiles = 10
    n_per_tile = n // nz_tiles
    p_per_tile = p // NW
    n_chunks = p_per_tile // chunk
    assert n_chunks % 2 == 1
    mesh = plsc.VectorSubcoreMesh(core_axis_name="c", subcore_axis_name="s")

    @functools.partial(
        pl.kernel,
        out_type=jax.ShapeDtypeStruct((NC, n, d), jnp.float32),
        mesh=mesh,
        scratch_types=[
            pltpu.VMEM((p_per_tile,), jnp.int32),
            pltpu.VMEM((2, chunk), jnp.int32),
            pltpu.VMEM((chunk, d), jnp.float32),
            pltpu.VMEM((chunk, d), jnp.float32),
            pltpu.VMEM((chunk // 2, d), jnp.int32),
            pltpu.VMEM((chunk // 2, d), jnp.int32),
            pltpu.VMEM_SHARED((n, d), jnp.float32),
            pltpu.SemaphoreType.DMA,
            pltpu.SemaphoreType.DMA,
            pltpu.SemaphoreType.DMA,
            pltpu.SemaphoreType.DMA,
            pltpu.SemaphoreType.DMA,
            pltpu.SemaphoreType.DMA,
            pltpu.SemaphoreType.DMA,
            pltpu.SemaphoreType.DMA,
        ],
    )
    def sc_kernel(h_hbm, wij_hbm, idxj_hbm, idxi_hbm, zero_hbm, out_hbm,
                  idxj_v, sidx_v, rows0, rows1, wij0, wij1, acc_sh,
                  gsem0, gsem1, wsem0, wsem1, ssem0, ssem1, isem0, isem1):
        rows = (rows0, rows1)
        wij = (wij0, wij1)
        gsem = (gsem0, gsem1)
        wsem = (wsem0, wsem1)
        ssem = (ssem0, ssem1)
        isem = (isem0, isem1)
        c = lax.axis_index("c")
        s = lax.axis_index("s")
        wid = c * NS + s

        row0 = s * n_per_tile

        @pl.when(s < nz_tiles)
        def _():
            pltpu.sync_copy(zero_hbm.at[pl.ds(row0, n_per_tile)],
                            acc_sh.at[pl.ds(row0, n_per_tile)])

        base0 = wid * p_per_tile
        pltpu.sync_copy(idxj_hbm.at[pl.ds(base0, p_per_tile)], idxj_v)
        plsc.subcore_barrier()

        rpc = chunk // 2
        wij_base = wid * (p_per_tile // 2)

        def issue(j, b):
            pltpu.async_copy(h_hbm.at[idxj_v.at[pl.ds(j * chunk, chunk)]],
                             rows[b], gsem[b])
            pltpu.async_copy(wij_hbm.at[pl.ds(wij_base + j * rpc, rpc)],
                             wij[b], wsem[b])
            pltpu.async_copy(idxi_hbm.at[pl.ds(base0 + j * chunk, chunk)],
                             sidx_v.at[b], isem[b])

        def step(j, b):
            nb = 1 - b

            @pl.when(j + 1 < n_chunks)
            def _():
                @pl.when(j >= 1)
                def _():
                    pltpu.make_async_copy(
                        rows[nb], acc_sh.at[sidx_v.at[nb]], ssem[nb]).wait()
                issue(j + 1, nb)

            pltpu.make_async_copy(h_hbm.at[idxj_v.at[pl.ds(0, chunk)]],
                                  rows[b], gsem[b]).wait()
            pltpu.make_async_copy(wij_hbm.at[pl.ds(0, rpc)], wij[b],
                                  wsem[b]).wait()

            def mul_body(rr2, carry):
                for par in range(2):
                    row = 2 * rr2 + par
                    for g in range(d // 32):
                        u = lax.bitcast_convert_type(
                            wij[b][rr2, pl.ds(64 * par + 16 * g, 16)],
                            jnp.uint32)
                        lo = lax.bitcast_convert_type(u << 16, jnp.float32)
                        hi = lax.bitcast_convert_type(
                            u & jnp.uint32(0xFFFF0000), jnp.float32)
                        sl0 = pl.ds(32 * g, LANES)
                        sl1 = pl.ds(32 * g + LANES, LANES)
                        rows[b][row, sl0] = rows[b][row, sl0] * lo
                        rows[b][row, sl1] = rows[b][row, sl1] * hi
                return carry

            lax.fori_loop(0, chunk // 2, mul_body, 0)
            pltpu.make_async_copy(idxi_hbm.at[pl.ds(0, chunk)], sidx_v.at[b],
                                  isem[b]).wait()
            pltpu.async_copy(rows[b], acc_sh.at[sidx_v.at[b]], ssem[b], add=True)

        issue(0, 0)

        def pair_body(t, carry):
            step(2 * t, 0)
            step(2 * t + 1, 1)
            return carry

        lax.fori_loop(0, (n_chunks - 1) // 2, pair_body, 0)
        step(n_chunks - 1, 0)
        pltpu.make_async_copy(rows[1], acc_sh.at[sidx_v.at[1]], ssem[1]).wait()
        pltpu.make_async_copy(rows[0], acc_sh.at[sidx_v.at[0]], ssem[0]).wait()
        plsc.subcore_barrier()

        @pl.when(s < nz_tiles)
        def _():
            pltpu.sync_copy(acc_sh.at[pl.ds(row0, n_per_tile)],
                            out_hbm.at[c, pl.ds(row0, n_per_tile)])

    return sc_kernel


def kernel(x, f_ij, idx_i, idx_j, rcut_ij, W1, b1, Wf, bf, W2, b2):
    n, d = x.shape
    p = f_ij.shape[0]
    chunk = 80
    h = _compute_h(x, W1, b1)
    wij = _compute_wij(f_ij, Wf, bf, rcut_ij)
    zeros = jnp.zeros((n, d), jnp.float32)
    idxj_in = jnp.stack([idx_j[:p // 2], idx_j[p // 2:]], axis=1).reshape(p)
    idxi_in = jnp.stack([idx_i[:p // 2], idx_i[p // 2:]], axis=1).reshape(p)
    sc = _make_sc_scatter(n, d, p, chunk=chunk)
    parts = sc(h, wij, idxj_in.astype(jnp.int32), idxi_in.astype(jnp.int32),
               zeros)
    return _compute_out(parts, W2, b2)


# speedup vs baseline: 1.6525x; 1.6525x over previous
import functools

import jax
import jax.numpy as jnp
import numpy as _np
from jax import lax
from jax.experimental import pallas as pl
from jax.experimental.pallas import tpu as pltpu
from jax.experimental.pallas import tpu_sc as plsc

NC = 2
NS = 16
NW = NC * NS
LANES = 16

def _interleave_perm(f):
    perm = _np.empty((f,), dtype=_np.int32)
    for g in range(f // 32):
        for l in range(16):
            perm[32 * g + 2 * l] = 32 * g + l
            perm[32 * g + 2 * l + 1] = 32 * g + 16 + l
    return perm


def _ssp(v):
    return jnp.maximum(v, 0.0) + jnp.log1p(jnp.exp(-jnp.abs(v))) - 0.6931471805599453


def _h_body(x_ref, w1t_ref, b1_ref, o_ref):
    o_ref[...] = jnp.dot(x_ref[...], w1t_ref[...],
                         preferred_element_type=jnp.float32) + b1_ref[...]


def _compute_h(x, W1, b1):
    n, d = x.shape
    blk = 1000
    grid = n // blk
    return pl.pallas_call(
        _h_body,
        grid=(grid,),
        in_specs=[
            pl.BlockSpec((blk, d), lambda i: (i, 0)),
            pl.BlockSpec((d, W1.shape[0]), lambda i: (0, 0)),
            pl.BlockSpec((1, W1.shape[0]), lambda i: (0, 0)),
        ],
        out_specs=pl.BlockSpec((blk, W1.shape[0]), lambda i: (i, 0)),
        out_shape=jax.ShapeDtypeStruct((n, W1.shape[0]), jnp.float32),
    )(x, W1.T, b1.reshape(1, -1))


def _pack_bf16_pair(v):
    f = v.shape[1]
    a = lax.bitcast_convert_type(v[:, :f // 2].astype(jnp.bfloat16),
                                 jnp.uint16).astype(jnp.uint32)
    b = lax.bitcast_convert_type(v[:, f // 2:].astype(jnp.bfloat16),
                                 jnp.uint16).astype(jnp.uint32)
    return lax.bitcast_convert_type(a | (b << 16), jnp.int32)


_DN = (((0,), (0,)), ((), ()))


def _wij_body(f1_ref, f2_ref, wft_ref, bf_ref, o_ref):
    v1 = _ssp(lax.dot_general(f1_ref[...], wft_ref[...], _DN,
                              preferred_element_type=jnp.float32) + bf_ref[...])
    v2 = _ssp(lax.dot_general(f2_ref[...], wft_ref[...], _DN,
                              preferred_element_type=jnp.float32) + bf_ref[...])
    o_ref[...] = jnp.concatenate([_pack_bf16_pair(v1), _pack_bf16_pair(v2)],
                                 axis=1)


def _compute_wij(f_ij, Wf, bf):
    p, r = f_ij.shape
    f = Wf.shape[0]
    perm = _interleave_perm(f)
    perm2 = _np.concatenate([perm[0::2], perm[1::2]])
    blk = 1280
    grid = (p // 2) // blk
    ft = f_ij.T
    return pl.pallas_call(
        _wij_body,
        grid=(grid,),
        in_specs=[
            pl.BlockSpec((r, blk), lambda i: (0, i)),
            pl.BlockSpec((r, blk), lambda i: (0, i + grid)),
            pl.BlockSpec((r, f), lambda i: (0, 0)),
            pl.BlockSpec((1, f), lambda i: (0, 0)),
        ],
        out_specs=pl.BlockSpec((blk, f), lambda i: (i, 0)),
        out_shape=jax.ShapeDtypeStruct((p // 2, f), jnp.int32),
    )(ft, ft, Wf.T[:, perm2], bf[perm2].reshape(1, -1))


def _out_body(p_ref, w2t_ref, b2_ref, o_ref):
    acc = p_ref[0] + p_ref[1]
    o_ref[...] = _ssp(jnp.dot(acc, w2t_ref[...],
                              preferred_element_type=jnp.float32) + b2_ref[...])


def _compute_out(parts, W2, b2):
    _, n, f = parts.shape
    d = W2.shape[0]
    blk = 1000
    grid = n // blk
    return pl.pallas_call(
        _out_body,
        grid=(grid,),
        in_specs=[
            pl.BlockSpec((2, blk, f), lambda i: (0, i, 0)),
            pl.BlockSpec((f, d), lambda i: (0, 0)),
            pl.BlockSpec((1, d), lambda i: (0, 0)),
        ],
        out_specs=pl.BlockSpec((blk, d), lambda i: (i, 0)),
        out_shape=jax.ShapeDtypeStruct((n, d), jnp.float32),
    )(parts, W2.T, b2.reshape(1, -1))


def _make_sc_scatter(n, d, p, chunk):
    nz_tiles = 10
    n_per_tile = n // nz_tiles
    half = chunk // 2
    p_half_tile = (p // 2) // NW
    n_chunks = p_half_tile // half
    assert n_chunks % 2 == 1
    mesh = plsc.VectorSubcoreMesh(core_axis_name="c", subcore_axis_name="s")

    @functools.partial(
        pl.kernel,
        out_type=jax.ShapeDtypeStruct((NC, n, d), jnp.float32),
        mesh=mesh,
        scratch_types=[
            pltpu.VMEM((2 * p_half_tile,), jnp.int32),
            pltpu.VMEM((2, half), jnp.int32),
            pltpu.VMEM((2, half), jnp.int32),
            pltpu.VMEM((2, half + LANES), jnp.float32),
            pltpu.VMEM((2, half + LANES), jnp.float32),
            pltpu.VMEM((chunk, d), jnp.float32),
            pltpu.VMEM((chunk, d), jnp.float32),
            pltpu.VMEM((half, d), jnp.int32),
            pltpu.VMEM((half, d), jnp.int32),
            pltpu.VMEM_SHARED((n, d), jnp.float32),
            pltpu.SemaphoreType.DMA,
            pltpu.SemaphoreType.DMA,
            pltpu.SemaphoreType.DMA,
            pltpu.SemaphoreType.DMA,
            pltpu.SemaphoreType.DMA,
            pltpu.SemaphoreType.DMA,
            pltpu.SemaphoreType.DMA,
            pltpu.SemaphoreType.DMA,
        ],
    )
    def sc_kernel(h_hbm, wij_hbm, idxj_hbm, idxi_hbm, rc_hbm, zero_hbm, out_hbm,
                  idxj_v, silo_v, sihi_v, rclo_v, rchi_v, rows0, rows1,
                  wij0, wij1, acc_sh,
                  gsem0, gsem1, wsem0, wsem1, ssem0, ssem1, isem0, isem1):
        rows = (rows0, rows1)
        wij = (wij0, wij1)
        gsem = (gsem0, gsem1)
        wsem = (wsem0, wsem1)
        ssem = (ssem0, ssem1)
        isem = (isem0, isem1)
        c = lax.axis_index("c")
        s = lax.axis_index("s")
        wid = c * NS + s

        row0 = s * n_per_tile

        @pl.when(s < nz_tiles)
        def _():
            pltpu.sync_copy(zero_hbm.at[pl.ds(row0, n_per_tile)],
                            acc_sh.at[pl.ds(row0, n_per_tile)])

        lo0 = wid * p_half_tile
        hi0 = (p // 2) + wid * p_half_tile
        pltpu.sync_copy(idxj_hbm.at[pl.ds(lo0, p_half_tile)],
                        idxj_v.at[pl.ds(0, p_half_tile)])
        pltpu.sync_copy(idxj_hbm.at[pl.ds(hi0, p_half_tile)],
                        idxj_v.at[pl.ds(p_half_tile, p_half_tile)])
        plsc.subcore_barrier()

        def issue(j, b):
            off = j * half
            pltpu.async_copy(h_hbm.at[idxj_v.at[pl.ds(off, half)]],
                             rows[b].at[pl.ds(0, half)], gsem[b])
            pltpu.async_copy(h_hbm.at[idxj_v.at[pl.ds(p_half_tile + off, half)]],
                             rows[b].at[pl.ds(half, half)], gsem[b])
            pltpu.async_copy(wij_hbm.at[pl.ds(lo0 + off, half)], wij[b], wsem[b])
            pltpu.async_copy(idxi_hbm.at[pl.ds(lo0 + off, half)],
                             silo_v.at[b], isem[b])
            pltpu.async_copy(idxi_hbm.at[pl.ds(hi0 + off, half)],
                             sihi_v.at[b], isem[b])
            pltpu.async_copy(rc_hbm.at[pl.ds(lo0 + off, half)],
                             rclo_v.at[b, pl.ds(0, half)], isem[b])
            pltpu.async_copy(rc_hbm.at[pl.ds(hi0 + off, half)],
                             rchi_v.at[b, pl.ds(0, half)], isem[b])

        def drain_scatter(b):
            pltpu.make_async_copy(rows[b].at[pl.ds(0, half)],
                                  acc_sh.at[silo_v.at[b]], ssem[b]).wait()
            pltpu.make_async_copy(rows[b].at[pl.ds(half, half)],
                                  acc_sh.at[sihi_v.at[b]], ssem[b]).wait()

        def step(j, b):
            nb = 1 - b

            @pl.when(j + 1 < n_chunks)
            def _():
                @pl.when(j >= 1)
                def _():
                    drain_scatter(nb)
                issue(j + 1, nb)

            pltpu.make_async_copy(h_hbm.at[idxj_v.at[pl.ds(0, half)]],
                                  rows[b].at[pl.ds(0, half)], gsem[b]).wait()
            pltpu.make_async_copy(h_hbm.at[idxj_v.at[pl.ds(0, half)]],
                                  rows[b].at[pl.ds(half, half)], gsem[b]).wait()
            pltpu.make_async_copy(wij_hbm.at[pl.ds(0, half)], wij[b],
                                  wsem[b]).wait()
            for _ in range(4):
                pltpu.make_async_copy(rc_hbm.at[pl.ds(0, half)],
                                      rclo_v.at[b, pl.ds(0, half)],
                                      isem[b]).wait()
            zero16 = jnp.zeros((LANES, 1), jnp.int32)
            gdn = lax.GatherDimensionNumbers(offset_dims=(),
                                             collapsed_slice_dims=(0,),
                                             start_index_map=(0,))

            def mul_body(rr2, carry):
                for par, rowoff, rc_v in ((0, 0, rclo_v), (1, half, rchi_v)):
                    row = rowoff + rr2
                    rcv = rc_v[b, pl.ds(rr2, LANES)]
                    scale = lax.gather(
                        rcv, zero16, gdn, (1,),
                        mode=lax.GatherScatterMode.PROMISE_IN_BOUNDS)
                    for g in range(d // 32):
                        u = lax.bitcast_convert_type(
                            wij[b][rr2, pl.ds(64 * par + 16 * g, 16)],
                            jnp.uint32)
                        lo = lax.bitcast_convert_type(u << 16, jnp.float32) * scale
                        hi = lax.bitcast_convert_type(
                            u & jnp.uint32(0xFFFF0000), jnp.float32) * scale
                        sl0 = pl.ds(32 * g, LANES)
                        sl1 = pl.ds(32 * g + LANES, LANES)
                        rows[b][row, sl0] = rows[b][row, sl0] * lo
                        rows[b][row, sl1] = rows[b][row, sl1] * hi
                return carry

            lax.fori_loop(0, half, mul_body, 0)
            pltpu.async_copy(rows[b].at[pl.ds(0, half)],
                             acc_sh.at[silo_v.at[b]], ssem[b], add=True)
            pltpu.async_copy(rows[b].at[pl.ds(half, half)],
                             acc_sh.at[sihi_v.at[b]], ssem[b], add=True)

        issue(0, 0)

        def pair_body(t, carry):
            step(2 * t, 0)
            step(2 * t + 1, 1)
            return carry

        lax.fori_loop(0, (n_chunks - 1) // 2, pair_body, 0)
        step(n_chunks - 1, 0)
        drain_scatter(1)
        drain_scatter(0)
        plsc.subcore_barrier()

        @pl.when(s < nz_tiles)
        def _():
            pltpu.sync_copy(acc_sh.at[pl.ds(row0, n_per_tile)],
                            out_hbm.at[c, pl.ds(row0, n_per_tile)])

    return sc_kernel


def kernel(x, f_ij, idx_i, idx_j, rcut_ij, W1, b1, Wf, bf, W2, b2):
    n, d = x.shape
    p = f_ij.shape[0]
    chunk = 80
    h = _compute_h(x, W1, b1)
    wij = _compute_wij(f_ij, Wf, bf)
    zeros = jnp.zeros((n, d), jnp.float32)
    sc = _make_sc_scatter(n, d, p, chunk=chunk)
    parts = sc(h, wij, idx_j.astype(jnp.int32), idx_i.astype(jnp.int32),
               rcut_ij, zeros)
    return _compute_out(parts, W2, b2)


# speedup vs baseline: 1.7122x; 1.0361x over previous
import functools

import jax
import jax.numpy as jnp
import numpy as _np
from jax import lax
from jax.experimental import pallas as pl
from jax.experimental.pallas import tpu as pltpu
from jax.experimental.pallas import tpu_sc as plsc

NC = 2
NS = 16
NW = NC * NS
LANES = 16

def _interleave_perm(f):
    perm = _np.empty((f,), dtype=_np.int32)
    for g in range(f // 32):
        for l in range(16):
            perm[32 * g + 2 * l] = 32 * g + l
            perm[32 * g + 2 * l + 1] = 32 * g + 16 + l
    return perm


def _ssp(v):
    return jnp.maximum(v, 0.0) + jnp.log1p(jnp.exp(-jnp.abs(v))) - 0.6931471805599453


_LOG2E = 1.4426950408889634
_LN2 = 0.6931471805599453


def _ssp_fast(v):
    t = v * _LOG2E
    tc = jnp.minimum(t, 120.0)
    return _LN2 * (jnp.log2(1.0 + jnp.exp2(tc)) - 1.0 + jnp.maximum(t - 120.0, 0.0))


def _h_body(x_ref, w1t_ref, b1_ref, o_ref):
    o_ref[...] = jnp.dot(x_ref[...], w1t_ref[...],
                         preferred_element_type=jnp.float32) + b1_ref[...]


def _compute_h(x, W1, b1):
    n, d = x.shape
    blk = 1000
    grid = n // blk
    return pl.pallas_call(
        _h_body,
        grid=(grid,),
        in_specs=[
            pl.BlockSpec((blk, d), lambda i: (i, 0)),
            pl.BlockSpec((d, W1.shape[0]), lambda i: (0, 0)),
            pl.BlockSpec((1, W1.shape[0]), lambda i: (0, 0)),
        ],
        out_specs=pl.BlockSpec((blk, W1.shape[0]), lambda i: (i, 0)),
        out_shape=jax.ShapeDtypeStruct((n, W1.shape[0]), jnp.float32),
    )(x, W1.T, b1.reshape(1, -1))


def _pack_bf16_pair(v):
    f = v.shape[1]
    a = lax.bitcast_convert_type(v[:, :f // 2].astype(jnp.bfloat16),
                                 jnp.uint16).astype(jnp.uint32)
    b = lax.bitcast_convert_type(v[:, f // 2:].astype(jnp.bfloat16),
                                 jnp.uint16).astype(jnp.uint32)
    return lax.bitcast_convert_type(a | (b << 16), jnp.int32)


_DN = (((0,), (0,)), ((), ()))


def _wij_body(f1_ref, f2_ref, wft_ref, bf_ref, o_ref):
    v1 = _ssp_fast(lax.dot_general(f1_ref[...], wft_ref[...], _DN,
                                   preferred_element_type=jnp.float32)
                   + bf_ref[...])
    v2 = _ssp_fast(lax.dot_general(f2_ref[...], wft_ref[...], _DN,
                                   preferred_element_type=jnp.float32)
                   + bf_ref[...])
    o_ref[...] = jnp.concatenate([_pack_bf16_pair(v1), _pack_bf16_pair(v2)],
                                 axis=1)


def _compute_wij(f_ij, Wf, bf):
    p, r = f_ij.shape
    f = Wf.shape[0]
    perm = _interleave_perm(f)
    perm2 = _np.concatenate([perm[0::2], perm[1::2]])
    blk = 1280
    grid = (p // 2) // blk
    ft = f_ij.T
    return pl.pallas_call(
        _wij_body,
        grid=(grid,),
        in_specs=[
            pl.BlockSpec((r, blk), lambda i: (0, i)),
            pl.BlockSpec((r, blk), lambda i: (0, i + grid)),
            pl.BlockSpec((r, f), lambda i: (0, 0)),
            pl.BlockSpec((1, f), lambda i: (0, 0)),
        ],
        out_specs=pl.BlockSpec((blk, f), lambda i: (i, 0)),
        out_shape=jax.ShapeDtypeStruct((p // 2, f), jnp.int32),
    )(ft, ft, Wf.T[:, perm2], bf[perm2].reshape(1, -1))


def _out_body(p_ref, w2t_ref, b2_ref, o_ref):
    acc = p_ref[0] + p_ref[1]
    o_ref[...] = _ssp(jnp.dot(acc, w2t_ref[...],
                              preferred_element_type=jnp.float32) + b2_ref[...])


def _compute_out(parts, W2, b2):
    _, n, f = parts.shape
    d = W2.shape[0]
    blk = 1000
    grid = n // blk
    return pl.pallas_call(
        _out_body,
        grid=(grid,),
        in_specs=[
            pl.BlockSpec((2, blk, f), lambda i: (0, i, 0)),
            pl.BlockSpec((f, d), lambda i: (0, 0)),
            pl.BlockSpec((1, d), lambda i: (0, 0)),
        ],
        out_specs=pl.BlockSpec((blk, d), lambda i: (i, 0)),
        out_shape=jax.ShapeDtypeStruct((n, d), jnp.float32),
    )(parts, W2.T, b2.reshape(1, -1))


def _make_sc_scatter(n, d, p, chunk):
    nz_tiles = 10
    n_per_tile = n // nz_tiles
    half = chunk // 2
    p_half_tile = (p // 2) // NW
    n_chunks = p_half_tile // half
    assert n_chunks % 2 == 1
    mesh = plsc.VectorSubcoreMesh(core_axis_name="c", subcore_axis_name="s")

    @functools.partial(
        pl.kernel,
        out_type=jax.ShapeDtypeStruct((NC, n, d), jnp.float32),
        mesh=mesh,
        scratch_types=[
            pltpu.VMEM((2 * p_half_tile,), jnp.int32),
            pltpu.VMEM((2, half), jnp.int32),
            pltpu.VMEM((2, half), jnp.int32),
            pltpu.VMEM((2, half + LANES), jnp.float32),
            pltpu.VMEM((2, half + LANES), jnp.float32),
            pltpu.VMEM((chunk, d), jnp.float32),
            pltpu.VMEM((chunk, d), jnp.float32),
            pltpu.VMEM((half, d), jnp.int32),
            pltpu.VMEM((half, d), jnp.int32),
            pltpu.VMEM_SHARED((n, d), jnp.float32),
            pltpu.SemaphoreType.DMA,
            pltpu.SemaphoreType.DMA,
            pltpu.SemaphoreType.DMA,
            pltpu.SemaphoreType.DMA,
            pltpu.SemaphoreType.DMA,
            pltpu.SemaphoreType.DMA,
            pltpu.SemaphoreType.DMA,
            pltpu.SemaphoreType.DMA,
        ],
    )
    def sc_kernel(h_hbm, wij_hbm, idxj_hbm, idxi_hbm, rc_hbm, zero_hbm, out_hbm,
                  idxj_v, silo_v, sihi_v, rclo_v, rchi_v, rows0, rows1,
                  wij0, wij1, acc_sh,
                  gsem0, gsem1, wsem0, wsem1, ssem0, ssem1, isem0, isem1):
        rows = (rows0, rows1)
        wij = (wij0, wij1)
        gsem = (gsem0, gsem1)
        wsem = (wsem0, wsem1)
        ssem = (ssem0, ssem1)
        isem = (isem0, isem1)
        c = lax.axis_index("c")
        s = lax.axis_index("s")
        wid = c * NS + s

        row0 = s * n_per_tile

        @pl.when(s < nz_tiles)
        def _():
            pltpu.sync_copy(zero_hbm.at[pl.ds(row0, n_per_tile)],
                            acc_sh.at[pl.ds(row0, n_per_tile)])

        lo0 = wid * p_half_tile
        hi0 = (p // 2) + wid * p_half_tile
        pltpu.sync_copy(idxj_hbm.at[pl.ds(lo0, p_half_tile)],
                        idxj_v.at[pl.ds(0, p_half_tile)])
        pltpu.sync_copy(idxj_hbm.at[pl.ds(hi0, p_half_tile)],
                        idxj_v.at[pl.ds(p_half_tile, p_half_tile)])
        plsc.subcore_barrier()

        def issue(j, b):
            off = j * half
            pltpu.async_copy(h_hbm.at[idxj_v.at[pl.ds(off, half)]],
                             rows[b].at[pl.ds(0, half)], gsem[b])
            pltpu.async_copy(h_hbm.at[idxj_v.at[pl.ds(p_half_tile + off, half)]],
                             rows[b].at[pl.ds(half, half)], gsem[b])
            pltpu.async_copy(wij_hbm.at[pl.ds(lo0 + off, half)], wij[b], wsem[b])
            pltpu.async_copy(idxi_hbm.at[pl.ds(lo0 + off, half)],
                             silo_v.at[b], isem[b])
            pltpu.async_copy(idxi_hbm.at[pl.ds(hi0 + off, half)],
                             sihi_v.at[b], isem[b])
            pltpu.async_copy(rc_hbm.at[pl.ds(lo0 + off, half)],
                             rclo_v.at[b, pl.ds(0, half)], isem[b])
            pltpu.async_copy(rc_hbm.at[pl.ds(hi0 + off, half)],
                             rchi_v.at[b, pl.ds(0, half)], isem[b])

        def drain_scatter(b):
            pltpu.make_async_copy(rows[b].at[pl.ds(0, half)],
                                  acc_sh.at[silo_v.at[b]], ssem[b]).wait()
            pltpu.make_async_copy(rows[b].at[pl.ds(half, half)],
                                  acc_sh.at[sihi_v.at[b]], ssem[b]).wait()

        def step(j, b):
            nb = 1 - b

            @pl.when(j + 1 < n_chunks)
            def _():
                @pl.when(j >= 1)
                def _():
                    drain_scatter(nb)
                issue(j + 1, nb)

            pltpu.make_async_copy(h_hbm.at[idxj_v.at[pl.ds(0, half)]],
                                  rows[b].at[pl.ds(0, half)], gsem[b]).wait()
            pltpu.make_async_copy(h_hbm.at[idxj_v.at[pl.ds(0, half)]],
                                  rows[b].at[pl.ds(half, half)], gsem[b]).wait()
            pltpu.make_async_copy(wij_hbm.at[pl.ds(0, half)], wij[b],
                                  wsem[b]).wait()
            for _ in range(4):
                pltpu.make_async_copy(rc_hbm.at[pl.ds(0, half)],
                                      rclo_v.at[b, pl.ds(0, half)],
                                      isem[b]).wait()
            zero16 = jnp.zeros((LANES, 1), jnp.int32)
            gdn = lax.GatherDimensionNumbers(offset_dims=(),
                                             collapsed_slice_dims=(0,),
                                             start_index_map=(0,))

            def mul_body(rr2, carry):
                for par, rowoff, rc_v in ((0, 0, rclo_v), (1, half, rchi_v)):
                    row = rowoff + rr2
                    rcv = rc_v[b, pl.ds(rr2, LANES)]
                    scale = lax.gather(
                        rcv, zero16, gdn, (1,),
                        mode=lax.GatherScatterMode.PROMISE_IN_BOUNDS)
                    for g in range(d // 32):
                        u = lax.bitcast_convert_type(
                            wij[b][rr2, pl.ds(64 * par + 16 * g, 16)],
                            jnp.uint32)
                        lo = lax.bitcast_convert_type(u << 16, jnp.float32) * scale
                        hi = lax.bitcast_convert_type(
                            u & jnp.uint32(0xFFFF0000), jnp.float32) * scale
                        sl0 = pl.ds(32 * g, LANES)
                        sl1 = pl.ds(32 * g + LANES, LANES)
                        rows[b][row, sl0] = rows[b][row, sl0] * lo
                        rows[b][row, sl1] = rows[b][row, sl1] * hi
                return carry

            lax.fori_loop(0, half, mul_body, 0)
            pltpu.async_copy(rows[b].at[pl.ds(0, half)],
                             acc_sh.at[silo_v.at[b]], ssem[b], add=True)
            pltpu.async_copy(rows[b].at[pl.ds(half, half)],
                             acc_sh.at[sihi_v.at[b]], ssem[b], add=True)

        issue(0, 0)

        def pair_body(t, carry):
            step(2 * t, 0)
            step(2 * t + 1, 1)
            return carry

        lax.fori_loop(0, (n_chunks - 1) // 2, pair_body, 0)
        step(n_chunks - 1, 0)
        drain_scatter(1)
        drain_scatter(0)
        plsc.subcore_barrier()

        @pl.when(s < nz_tiles)
        def _():
            pltpu.sync_copy(acc_sh.at[pl.ds(row0, n_per_tile)],
                            out_hbm.at[c, pl.ds(row0, n_per_tile)])

    return sc_kernel


def kernel(x, f_ij, idx_i, idx_j, rcut_ij, W1, b1, Wf, bf, W2, b2):
    n, d = x.shape
    p = f_ij.shape[0]
    chunk = 80
    h = _compute_h(x, W1, b1)
    wij = _compute_wij(f_ij, Wf, bf)
    zeros = jnp.zeros((n, d), jnp.float32)
    sc = _make_sc_scatter(n, d, p, chunk=chunk)
    parts = sc(h, wij, idx_j.astype(jnp.int32), idx_i.astype(jnp.int32),
               rcut_ij, zeros)
    return _compute_out(parts, W2, b2)


# speedup vs baseline: 1.9014x; 1.1105x over previous
import functools

import jax
import jax.numpy as jnp
import numpy as _np
from jax import lax
from jax.experimental import pallas as pl
from jax.experimental.pallas import tpu as pltpu
from jax.experimental.pallas import tpu_sc as plsc

NC = 2
NS = 16
NW = NC * NS
LANES = 16

def _interleave_perm(f):
    perm = _np.empty((f,), dtype=_np.int32)
    for g in range(f // 32):
        for l in range(16):
            perm[32 * g + 2 * l] = 32 * g + l
            perm[32 * g + 2 * l + 1] = 32 * g + 16 + l
    return perm


def _ssp(v):
    return jnp.maximum(v, 0.0) + jnp.log1p(jnp.exp(-jnp.abs(v))) - 0.6931471805599453


_LOG2E = 1.4426950408889634
_LN2 = 0.6931471805599453


def _ssp_fast(v):
    t = v * _LOG2E
    tc = jnp.minimum(t, 120.0)
    return _LN2 * (jnp.log2(1.0 + jnp.exp2(tc)) - 1.0 + jnp.maximum(t - 120.0, 0.0))


def _h_body(x_ref, w1t_ref, b1_ref, o_ref):
    o_ref[...] = jnp.dot(x_ref[...], w1t_ref[...],
                         preferred_element_type=jnp.float32) + b1_ref[...]


def _compute_h(x, W1, b1):
    n, d = x.shape
    blk = 1000
    grid = n // blk
    return pl.pallas_call(
        _h_body,
        grid=(grid,),
        in_specs=[
            pl.BlockSpec((blk, d), lambda i: (i, 0)),
            pl.BlockSpec((d, W1.shape[0]), lambda i: (0, 0)),
            pl.BlockSpec((1, W1.shape[0]), lambda i: (0, 0)),
        ],
        out_specs=pl.BlockSpec((blk, W1.shape[0]), lambda i: (i, 0)),
        out_shape=jax.ShapeDtypeStruct((n, W1.shape[0]), jnp.float32),
    )(x, W1.T, b1.reshape(1, -1))


def _pack_bf16_pair(v):
    f = v.shape[1]
    a = lax.bitcast_convert_type(v[:, :f // 2].astype(jnp.bfloat16),
                                 jnp.uint16).astype(jnp.uint32)
    b = lax.bitcast_convert_type(v[:, f // 2:].astype(jnp.bfloat16),
                                 jnp.uint16).astype(jnp.uint32)
    return lax.bitcast_convert_type(a | (b << 16), jnp.int32)


_DN = (((0,), (0,)), ((), ()))


def _wij_body(f1_ref, f2_ref, wft_ref, bf_ref, o_ref):
    v1 = _ssp_fast(lax.dot_general(f1_ref[...], wft_ref[...], _DN,
                                   preferred_element_type=jnp.float32)
                   + bf_ref[...])
    v2 = _ssp_fast(lax.dot_general(f2_ref[...], wft_ref[...], _DN,
                                   preferred_element_type=jnp.float32)
                   + bf_ref[...])
    o_ref[...] = jnp.concatenate([_pack_bf16_pair(v1), _pack_bf16_pair(v2)],
                                 axis=1)


def _compute_wij(f_ij, Wf, bf, blk0, nblk):
    p, r = f_ij.shape
    f = Wf.shape[0]
    perm = _interleave_perm(f)
    perm2 = _np.concatenate([perm[0::2], perm[1::2]])
    blk = 1280
    half_blocks = (p // 2) // blk
    ft = f_ij.T
    return pl.pallas_call(
        _wij_body,
        grid=(nblk,),
        in_specs=[
            pl.BlockSpec((r, blk), lambda i: (0, i + blk0)),
            pl.BlockSpec((r, blk), lambda i: (0, i + blk0 + half_blocks)),
            pl.BlockSpec((r, f), lambda i: (0, 0)),
            pl.BlockSpec((1, f), lambda i: (0, 0)),
        ],
        out_specs=pl.BlockSpec((blk, f), lambda i: (i, 0)),
        out_shape=jax.ShapeDtypeStruct((nblk * blk, f), jnp.int32),
    )(ft, ft, Wf.T[:, perm2], bf[perm2].reshape(1, -1))


def _out_body(p_ref, w2t_ref, b2_ref, o_ref):
    acc = p_ref[0] + p_ref[1]
    o_ref[...] = _ssp(jnp.dot(acc, w2t_ref[...],
                              preferred_element_type=jnp.float32) + b2_ref[...])


def _compute_out(parts, W2, b2):
    _, n, f = parts.shape
    d = W2.shape[0]
    blk = 1000
    grid = n // blk
    return pl.pallas_call(
        _out_body,
        grid=(grid,),
        in_specs=[
            pl.BlockSpec((2, blk, f), lambda i: (0, i, 0)),
            pl.BlockSpec((f, d), lambda i: (0, 0)),
            pl.BlockSpec((1, d), lambda i: (0, 0)),
        ],
        out_specs=pl.BlockSpec((blk, d), lambda i: (i, 0)),
        out_shape=jax.ShapeDtypeStruct((n, d), jnp.float32),
    )(parts, W2.T, b2.reshape(1, -1))


def _make_sc_scatter(n, d, p, chunk, row_start, tile_rows):
    nz_tiles = 10
    n_per_tile = n // nz_tiles
    half = chunk // 2
    p_half_tile = tile_rows
    n_chunks = p_half_tile // half
    mesh = plsc.VectorSubcoreMesh(core_axis_name="c", subcore_axis_name="s")

    @functools.partial(
        pl.kernel,
        out_type=jax.ShapeDtypeStruct((NC, n, d), jnp.float32),
        mesh=mesh,
        scratch_types=[
            pltpu.VMEM((2 * p_half_tile,), jnp.int32),
            pltpu.VMEM((2, half), jnp.int32),
            pltpu.VMEM((2, half), jnp.int32),
            pltpu.VMEM((2, half + LANES), jnp.float32),
            pltpu.VMEM((2, half + LANES), jnp.float32),
            pltpu.VMEM((chunk, d), jnp.float32),
            pltpu.VMEM((chunk, d), jnp.float32),
            pltpu.VMEM((half, d), jnp.int32),
            pltpu.VMEM((half, d), jnp.int32),
            pltpu.VMEM_SHARED((n, d), jnp.float32),
            pltpu.SemaphoreType.DMA,
            pltpu.SemaphoreType.DMA,
            pltpu.SemaphoreType.DMA,
            pltpu.SemaphoreType.DMA,
            pltpu.SemaphoreType.DMA,
            pltpu.SemaphoreType.DMA,
            pltpu.SemaphoreType.DMA,
            pltpu.SemaphoreType.DMA,
        ],
    )
    def sc_kernel(h_hbm, wij_hbm, idxj_hbm, idxi_hbm, rc_hbm, zero_hbm, out_hbm,
                  idxj_v, silo_v, sihi_v, rclo_v, rchi_v, rows0, rows1,
                  wij0, wij1, acc_sh,
                  gsem0, gsem1, wsem0, wsem1, ssem0, ssem1, isem0, isem1):
        rows = (rows0, rows1)
        wij = (wij0, wij1)
        gsem = (gsem0, gsem1)
        wsem = (wsem0, wsem1)
        ssem = (ssem0, ssem1)
        isem = (isem0, isem1)
        c = lax.axis_index("c")
        s = lax.axis_index("s")
        wid = c * NS + s

        row0 = s * n_per_tile

        @pl.when(s < nz_tiles)
        def _():
            pltpu.sync_copy(zero_hbm.at[c, pl.ds(row0, n_per_tile)],
                            acc_sh.at[pl.ds(row0, n_per_tile)])

        lo0 = row_start + wid * p_half_tile
        hi0 = (p // 2) + row_start + wid * p_half_tile
        wloc0 = wid * p_half_tile
        pltpu.sync_copy(idxj_hbm.at[pl.ds(lo0, p_half_tile)],
                        idxj_v.at[pl.ds(0, p_half_tile)])
        pltpu.sync_copy(idxj_hbm.at[pl.ds(hi0, p_half_tile)],
                        idxj_v.at[pl.ds(p_half_tile, p_half_tile)])
        plsc.subcore_barrier()

        def issue(j, b):
            off = j * half
            pltpu.async_copy(h_hbm.at[idxj_v.at[pl.ds(off, half)]],
                             rows[b].at[pl.ds(0, half)], gsem[b])
            pltpu.async_copy(h_hbm.at[idxj_v.at[pl.ds(p_half_tile + off, half)]],
                             rows[b].at[pl.ds(half, half)], gsem[b])
            pltpu.async_copy(wij_hbm.at[pl.ds(wloc0 + off, half)], wij[b],
                             wsem[b])
            pltpu.async_copy(idxi_hbm.at[pl.ds(lo0 + off, half)],
                             silo_v.at[b], isem[b])
            pltpu.async_copy(idxi_hbm.at[pl.ds(hi0 + off, half)],
                             sihi_v.at[b], isem[b])
            pltpu.async_copy(rc_hbm.at[pl.ds(lo0 + off, half)],
                             rclo_v.at[b, pl.ds(0, half)], isem[b])
            pltpu.async_copy(rc_hbm.at[pl.ds(hi0 + off, half)],
                             rchi_v.at[b, pl.ds(0, half)], isem[b])

        def drain_scatter(b):
            pltpu.make_async_copy(rows[b].at[pl.ds(0, half)],
                                  acc_sh.at[silo_v.at[b]], ssem[b]).wait()
            pltpu.make_async_copy(rows[b].at[pl.ds(half, half)],
                                  acc_sh.at[sihi_v.at[b]], ssem[b]).wait()

        def step(j, b):
            nb = 1 - b

            @pl.when(j + 1 < n_chunks)
            def _():
                @pl.when(j >= 1)
                def _():
                    drain_scatter(nb)
                issue(j + 1, nb)

            pltpu.make_async_copy(h_hbm.at[idxj_v.at[pl.ds(0, half)]],
                                  rows[b].at[pl.ds(0, half)], gsem[b]).wait()
            pltpu.make_async_copy(h_hbm.at[idxj_v.at[pl.ds(0, half)]],
                                  rows[b].at[pl.ds(half, half)], gsem[b]).wait()
            pltpu.make_async_copy(wij_hbm.at[pl.ds(0, half)], wij[b],
                                  wsem[b]).wait()
            for _ in range(4):
                pltpu.make_async_copy(rc_hbm.at[pl.ds(0, half)],
                                      rclo_v.at[b, pl.ds(0, half)],
                                      isem[b]).wait()
            zero16 = jnp.zeros((LANES, 1), jnp.int32)
            gdn = lax.GatherDimensionNumbers(offset_dims=(),
                                             collapsed_slice_dims=(0,),
                                             start_index_map=(0,))

            def mul_body(rr2, carry):
                for par, rowoff, rc_v in ((0, 0, rclo_v), (1, half, rchi_v)):
                    row = rowoff + rr2
                    rcv = rc_v[b, pl.ds(rr2, LANES)]
                    scale = lax.gather(
                        rcv, zero16, gdn, (1,),
                        mode=lax.GatherScatterMode.PROMISE_IN_BOUNDS)
                    for g in range(d // 32):
                        u = lax.bitcast_convert_type(
                            wij[b][rr2, pl.ds(64 * par + 16 * g, 16)],
                            jnp.uint32)
                        lo = lax.bitcast_convert_type(u << 16, jnp.float32) * scale
                        hi = lax.bitcast_convert_type(
                            u & jnp.uint32(0xFFFF0000), jnp.float32) * scale
                        sl0 = pl.ds(32 * g, LANES)
                        sl1 = pl.ds(32 * g + LANES, LANES)
                        rows[b][row, sl0] = rows[b][row, sl0] * lo
                        rows[b][row, sl1] = rows[b][row, sl1] * hi
                return carry

            lax.fori_loop(0, half, mul_body, 0)
            pltpu.async_copy(rows[b].at[pl.ds(0, half)],
                             acc_sh.at[silo_v.at[b]], ssem[b], add=True)
            pltpu.async_copy(rows[b].at[pl.ds(half, half)],
                             acc_sh.at[sihi_v.at[b]], ssem[b], add=True)

        issue(0, 0)

        def pair_body(t, carry):
            step(2 * t, 0)
            step(2 * t + 1, 1)
            return carry

        lax.fori_loop(0, n_chunks // 2, pair_body, 0)
        if n_chunks % 2 == 1:
            step(n_chunks - 1, 0)
        drain_scatter(1)
        drain_scatter(0)
        plsc.subcore_barrier()

        @pl.when(s < nz_tiles)
        def _():
            pltpu.sync_copy(acc_sh.at[pl.ds(row0, n_per_tile)],
                            out_hbm.at[c, pl.ds(row0, n_per_tile)])

    return sc_kernel


def kernel(x, f_ij, idx_i, idx_j, rcut_ij, W1, b1, Wf, bf, W2, b2):
    n, d = x.shape
    p = f_ij.shape[0]
    chunk = 80
    h = _compute_h(x, W1, b1)
    idxj = idx_j.astype(jnp.int32)
    idxi = idx_i.astype(jnp.int32)
    blk = 1280
    tile_rows_a, tile_rows_b = 2480, 2520
    nblk_a = tile_rows_a * NW // blk
    nblk_b = tile_rows_b * NW // blk
    wij_a = _compute_wij(f_ij, Wf, bf, 0, nblk_a)
    wij_b = _compute_wij(f_ij, Wf, bf, nblk_a, nblk_b)
    zeros = jnp.zeros((NC, n, d), jnp.float32)
    sc_a = _make_sc_scatter(n, d, p, chunk, 0, tile_rows_a)
    sc_b = _make_sc_scatter(n, d, p, chunk, tile_rows_a * NW, tile_rows_b)
    parts_a = sc_a(h, wij_a, idxj, idxi, rcut_ij, zeros)
    parts_b = sc_b(h, wij_b, idxj, idxi, rcut_ij, parts_a)
    return _compute_out(parts_b, W2, b2)


# speedup vs baseline: 1.9849x; 1.0440x over previous
import functools

import jax
import jax.numpy as jnp
import numpy as _np
from jax import lax
from jax.experimental import pallas as pl
from jax.experimental.pallas import tpu as pltpu
from jax.experimental.pallas import tpu_sc as plsc

NC = 2
NS = 16
NW = NC * NS
LANES = 16

def _interleave_perm(f):
    perm = _np.empty((f,), dtype=_np.int32)
    for g in range(f // 32):
        for l in range(16):
            perm[32 * g + 2 * l] = 32 * g + l
            perm[32 * g + 2 * l + 1] = 32 * g + 16 + l
    return perm


def _ssp(v):
    return jnp.maximum(v, 0.0) + jnp.log1p(jnp.exp(-jnp.abs(v))) - 0.6931471805599453


_LOG2E = 1.4426950408889634
_LN2 = 0.6931471805599453


def _ssp_fast(v):
    t = v * _LOG2E
    tc = jnp.minimum(t, 120.0)
    return _LN2 * (jnp.log2(1.0 + jnp.exp2(tc)) - 1.0 + jnp.maximum(t - 120.0, 0.0))


def _h_body(x_ref, w1t_ref, b1_ref, o_ref):
    o_ref[...] = jnp.dot(x_ref[...], w1t_ref[...],
                         preferred_element_type=jnp.float32) + b1_ref[...]


def _compute_h(x, W1, b1):
    n, d = x.shape
    blk = 1000
    grid = n // blk
    return pl.pallas_call(
        _h_body,
        grid=(grid,),
        in_specs=[
            pl.BlockSpec((blk, d), lambda i: (i, 0)),
            pl.BlockSpec((d, W1.shape[0]), lambda i: (0, 0)),
            pl.BlockSpec((1, W1.shape[0]), lambda i: (0, 0)),
        ],
        out_specs=pl.BlockSpec((blk, W1.shape[0]), lambda i: (i, 0)),
        out_shape=jax.ShapeDtypeStruct((n, W1.shape[0]), jnp.float32),
    )(x, W1.T, b1.reshape(1, -1))


def _pack_bf16_pair(v):
    f = v.shape[1]
    a = lax.bitcast_convert_type(v[:, :f // 2].astype(jnp.bfloat16),
                                 jnp.uint16).astype(jnp.uint32)
    b = lax.bitcast_convert_type(v[:, f // 2:].astype(jnp.bfloat16),
                                 jnp.uint16).astype(jnp.uint32)
    return lax.bitcast_convert_type(a | (b << 16), jnp.int32)


_DN = (((0,), (0,)), ((), ()))


def _wij_body(f1_ref, f2_ref, wft_ref, bf_ref, o_ref):
    v1 = _ssp_fast(lax.dot_general(f1_ref[...], wft_ref[...], _DN,
                                   preferred_element_type=jnp.float32)
                   + bf_ref[...])
    v2 = _ssp_fast(lax.dot_general(f2_ref[...], wft_ref[...], _DN,
                                   preferred_element_type=jnp.float32)
                   + bf_ref[...])
    o_ref[...] = jnp.concatenate([_pack_bf16_pair(v1), _pack_bf16_pair(v2)],
                                 axis=1)


def _compute_wij(f_ij, Wf, bf, blk0, nblk):
    p, r = f_ij.shape
    f = Wf.shape[0]
    perm = _interleave_perm(f)
    perm2 = _np.concatenate([perm[0::2], perm[1::2]])
    blk = 1280
    half_blocks = (p // 2) // blk
    ft = f_ij.T
    return pl.pallas_call(
        _wij_body,
        grid=(nblk,),
        in_specs=[
            pl.BlockSpec((r, blk), lambda i: (0, i + blk0)),
            pl.BlockSpec((r, blk), lambda i: (0, i + blk0 + half_blocks)),
            pl.BlockSpec((r, f), lambda i: (0, 0)),
            pl.BlockSpec((1, f), lambda i: (0, 0)),
        ],
        out_specs=pl.BlockSpec((blk, f), lambda i: (i, 0)),
        out_shape=jax.ShapeDtypeStruct((nblk * blk, f), jnp.int32),
    )(ft, ft, Wf.T[:, perm2], bf[perm2].reshape(1, -1))


def _out_body(p_ref, w2t_ref, b2_ref, o_ref):
    acc = p_ref[0] + p_ref[1]
    o_ref[...] = _ssp(jnp.dot(acc, w2t_ref[...],
                              preferred_element_type=jnp.float32) + b2_ref[...])


def _compute_out(parts, W2, b2):
    _, n, f = parts.shape
    d = W2.shape[0]
    blk = 1000
    grid = n // blk
    return pl.pallas_call(
        _out_body,
        grid=(grid,),
        in_specs=[
            pl.BlockSpec((2, blk, f), lambda i: (0, i, 0)),
            pl.BlockSpec((f, d), lambda i: (0, 0)),
            pl.BlockSpec((1, d), lambda i: (0, 0)),
        ],
        out_specs=pl.BlockSpec((blk, d), lambda i: (i, 0)),
        out_shape=jax.ShapeDtypeStruct((n, d), jnp.float32),
    )(parts, W2.T, b2.reshape(1, -1))


def _make_sc_scatter(n, d, p, chunk, row_start, tile_rows):
    nz_tiles = 10
    n_per_tile = n // nz_tiles
    half = chunk // 2
    p_half_tile = tile_rows
    n_chunks = p_half_tile // half
    mesh = plsc.VectorSubcoreMesh(core_axis_name="c", subcore_axis_name="s")

    @functools.partial(
        pl.kernel,
        out_type=jax.ShapeDtypeStruct((NC, n, d), jnp.float32),
        mesh=mesh,
        scratch_types=[
            pltpu.VMEM((2 * p_half_tile,), jnp.int32),
            pltpu.VMEM((2, half), jnp.int32),
            pltpu.VMEM((2, half), jnp.int32),
            pltpu.VMEM((2, half + LANES), jnp.float32),
            pltpu.VMEM((2, half + LANES), jnp.float32),
            pltpu.VMEM((chunk, d), jnp.float32),
            pltpu.VMEM((chunk, d), jnp.float32),
            pltpu.VMEM((half, d), jnp.int32),
            pltpu.VMEM((half, d), jnp.int32),
            pltpu.VMEM_SHARED((n, d), jnp.float32),
            pltpu.SemaphoreType.DMA,
            pltpu.SemaphoreType.DMA,
            pltpu.SemaphoreType.DMA,
            pltpu.SemaphoreType.DMA,
            pltpu.SemaphoreType.DMA,
            pltpu.SemaphoreType.DMA,
            pltpu.SemaphoreType.DMA,
            pltpu.SemaphoreType.DMA,
        ],
    )
    def sc_kernel(h_hbm, wij_hbm, idxj_hbm, idxi_hbm, rc_hbm, zero_hbm, out_hbm,
                  idxj_v, silo_v, sihi_v, rclo_v, rchi_v, rows0, rows1,
                  wij0, wij1, acc_sh,
                  gsem0, gsem1, wsem0, wsem1, ssem0, ssem1, isem0, isem1):
        rows = (rows0, rows1)
        wij = (wij0, wij1)
        gsem = (gsem0, gsem1)
        wsem = (wsem0, wsem1)
        ssem = (ssem0, ssem1)
        isem = (isem0, isem1)
        c = lax.axis_index("c")
        s = lax.axis_index("s")
        wid = c * NS + s

        row0 = s * n_per_tile

        @pl.when(s < nz_tiles)
        def _():
            pltpu.sync_copy(zero_hbm.at[c, pl.ds(row0, n_per_tile)],
                            acc_sh.at[pl.ds(row0, n_per_tile)])

        lo0 = row_start + wid * p_half_tile
        hi0 = (p // 2) + row_start + wid * p_half_tile
        wloc0 = wid * p_half_tile
        pltpu.sync_copy(idxj_hbm.at[pl.ds(lo0, p_half_tile)],
                        idxj_v.at[pl.ds(0, p_half_tile)])
        pltpu.sync_copy(idxj_hbm.at[pl.ds(hi0, p_half_tile)],
                        idxj_v.at[pl.ds(p_half_tile, p_half_tile)])
        plsc.subcore_barrier()

        def issue(j, b):
            off = j * half
            pltpu.async_copy(h_hbm.at[idxj_v.at[pl.ds(off, half)]],
                             rows[b].at[pl.ds(0, half)], gsem[b])
            pltpu.async_copy(h_hbm.at[idxj_v.at[pl.ds(p_half_tile + off, half)]],
                             rows[b].at[pl.ds(half, half)], gsem[b])
            pltpu.async_copy(wij_hbm.at[pl.ds(wloc0 + off, half)], wij[b],
                             wsem[b])
            pltpu.async_copy(idxi_hbm.at[pl.ds(lo0 + off, half)],
                             silo_v.at[b], isem[b])
            pltpu.async_copy(idxi_hbm.at[pl.ds(hi0 + off, half)],
                             sihi_v.at[b], isem[b])
            pltpu.async_copy(rc_hbm.at[pl.ds(lo0 + off, half)],
                             rclo_v.at[b, pl.ds(0, half)], isem[b])
            pltpu.async_copy(rc_hbm.at[pl.ds(hi0 + off, half)],
                             rchi_v.at[b, pl.ds(0, half)], isem[b])

        def drain_scatter(b):
            pltpu.make_async_copy(rows[b].at[pl.ds(0, half)],
                                  acc_sh.at[silo_v.at[b]], ssem[b]).wait()
            pltpu.make_async_copy(rows[b].at[pl.ds(half, half)],
                                  acc_sh.at[sihi_v.at[b]], ssem[b]).wait()

        def step(j, b):
            nb = 1 - b

            @pl.when(j + 1 < n_chunks)
            def _():
                @pl.when(j >= 1)
                def _():
                    drain_scatter(nb)
                issue(j + 1, nb)

            pltpu.make_async_copy(h_hbm.at[idxj_v.at[pl.ds(0, half)]],
                                  rows[b].at[pl.ds(0, half)], gsem[b]).wait()
            pltpu.make_async_copy(h_hbm.at[idxj_v.at[pl.ds(0, half)]],
                                  rows[b].at[pl.ds(half, half)], gsem[b]).wait()
            pltpu.make_async_copy(wij_hbm.at[pl.ds(0, half)], wij[b],
                                  wsem[b]).wait()
            for _ in range(4):
                pltpu.make_async_copy(rc_hbm.at[pl.ds(0, half)],
                                      rclo_v.at[b, pl.ds(0, half)],
                                      isem[b]).wait()
            zero16 = jnp.zeros((LANES, 1), jnp.int32)
            gdn = lax.GatherDimensionNumbers(offset_dims=(),
                                             collapsed_slice_dims=(0,),
                                             start_index_map=(0,))

            def mul_body(rr2, carry):
                for par, rowoff, rc_v in ((0, 0, rclo_v), (1, half, rchi_v)):
                    row = rowoff + rr2
                    rcv = rc_v[b, pl.ds(rr2, LANES)]
                    scale = lax.gather(
                        rcv, zero16, gdn, (1,),
                        mode=lax.GatherScatterMode.PROMISE_IN_BOUNDS)
                    for g in range(d // 32):
                        u = lax.bitcast_convert_type(
                            wij[b][rr2, pl.ds(64 * par + 16 * g, 16)],
                            jnp.uint32)
                        lo = lax.bitcast_convert_type(u << 16, jnp.float32) * scale
                        hi = lax.bitcast_convert_type(
                            u & jnp.uint32(0xFFFF0000), jnp.float32) * scale
                        sl0 = pl.ds(32 * g, LANES)
                        sl1 = pl.ds(32 * g + LANES, LANES)
                        rows[b][row, sl0] = rows[b][row, sl0] * lo
                        rows[b][row, sl1] = rows[b][row, sl1] * hi
                return carry

            lax.fori_loop(0, half, mul_body, 0)
            pltpu.async_copy(rows[b].at[pl.ds(0, half)],
                             acc_sh.at[silo_v.at[b]], ssem[b], add=True)
            pltpu.async_copy(rows[b].at[pl.ds(half, half)],
                             acc_sh.at[sihi_v.at[b]], ssem[b], add=True)

        issue(0, 0)

        def pair_body(t, carry):
            step(2 * t, 0)
            step(2 * t + 1, 1)
            return carry

        lax.fori_loop(0, n_chunks // 2, pair_body, 0)
        if n_chunks % 2 == 1:
            step(n_chunks - 1, 0)
        drain_scatter(1)
        drain_scatter(0)
        plsc.subcore_barrier()

        @pl.when(s < nz_tiles)
        def _():
            pltpu.sync_copy(acc_sh.at[pl.ds(row0, n_per_tile)],
                            out_hbm.at[c, pl.ds(row0, n_per_tile)])

    return sc_kernel


def kernel(x, f_ij, idx_i, idx_j, rcut_ij, W1, b1, Wf, bf, W2, b2):
    n, d = x.shape
    p = f_ij.shape[0]
    chunk = 80
    h = _compute_h(x, W1, b1)
    idxj = idx_j.astype(jnp.int32)
    idxi = idx_i.astype(jnp.int32)
    blk = 1280
    tile_rows_a, tile_rows_b = 1800, 3200
    nblk_a = tile_rows_a * NW // blk
    nblk_b = tile_rows_b * NW // blk
    wij_a = _compute_wij(f_ij, Wf, bf, 0, nblk_a)
    wij_b = _compute_wij(f_ij, Wf, bf, nblk_a, nblk_b)
    zeros = jnp.zeros((NC, n, d), jnp.float32)
    sc_a = _make_sc_scatter(n, d, p, chunk, 0, tile_rows_a)
    sc_b = _make_sc_scatter(n, d, p, chunk, tile_rows_a * NW, tile_rows_b)
    parts_a = sc_a(h, wij_a, idxj, idxi, rcut_ij, zeros)
    parts_b = sc_b(h, wij_b, idxj, idxi, rcut_ij, parts_a)
    return _compute_out(parts_b, W2, b2)


# speedup vs baseline: 2.0272x; 1.0213x over previous
import functools

import jax
import jax.numpy as jnp
import numpy as _np
from jax import lax
from jax.experimental import pallas as pl
from jax.experimental.pallas import tpu as pltpu
from jax.experimental.pallas import tpu_sc as plsc

NC = 2
NS = 16
NW = NC * NS
LANES = 16

def _interleave_perm(f):
    perm = _np.empty((f,), dtype=_np.int32)
    for g in range(f // 32):
        for l in range(16):
            perm[32 * g + 2 * l] = 32 * g + l
            perm[32 * g + 2 * l + 1] = 32 * g + 16 + l
    return perm


def _ssp(v):
    return jnp.maximum(v, 0.0) + jnp.log1p(jnp.exp(-jnp.abs(v))) - 0.6931471805599453


_LOG2E = 1.4426950408889634
_LN2 = 0.6931471805599453


def _ssp_fast(v):
    t = v * _LOG2E
    tc = jnp.minimum(t, 120.0)
    return _LN2 * (jnp.log2(1.0 + jnp.exp2(tc)) - 1.0 + jnp.maximum(t - 120.0, 0.0))


def _h_body(x_ref, w1t_ref, b1_ref, o_ref):
    o_ref[...] = jnp.dot(x_ref[...], w1t_ref[...],
                         preferred_element_type=jnp.float32) + b1_ref[...]


def _compute_h(x, W1, b1):
    n, d = x.shape
    blk = 1000
    grid = n // blk
    return pl.pallas_call(
        _h_body,
        grid=(grid,),
        in_specs=[
            pl.BlockSpec((blk, d), lambda i: (i, 0)),
            pl.BlockSpec((d, W1.shape[0]), lambda i: (0, 0)),
            pl.BlockSpec((1, W1.shape[0]), lambda i: (0, 0)),
        ],
        out_specs=pl.BlockSpec((blk, W1.shape[0]), lambda i: (i, 0)),
        out_shape=jax.ShapeDtypeStruct((n, W1.shape[0]), jnp.float32),
    )(x, W1.T, b1.reshape(1, -1))


def _pack_bf16_pair(v):
    f = v.shape[1]
    a = lax.bitcast_convert_type(v[:, :f // 2].astype(jnp.bfloat16),
                                 jnp.uint16).astype(jnp.uint32)
    b = lax.bitcast_convert_type(v[:, f // 2:].astype(jnp.bfloat16),
                                 jnp.uint16).astype(jnp.uint32)
    return lax.bitcast_convert_type(a | (b << 16), jnp.int32)


_DN = (((0,), (0,)), ((), ()))


def _wij_body(f1_ref, f2_ref, wft_ref, bf_ref, o_ref):
    v1 = _ssp_fast(lax.dot_general(f1_ref[...], wft_ref[...], _DN,
                                   preferred_element_type=jnp.float32)
                   + bf_ref[...])
    v2 = _ssp_fast(lax.dot_general(f2_ref[...], wft_ref[...], _DN,
                                   preferred_element_type=jnp.float32)
                   + bf_ref[...])
    o_ref[...] = jnp.concatenate([_pack_bf16_pair(v1), _pack_bf16_pair(v2)],
                                 axis=1)


def _compute_wij(f_ij, Wf, bf, blk0, nblk):
    p, r = f_ij.shape
    f = Wf.shape[0]
    perm = _interleave_perm(f)
    perm2 = _np.concatenate([perm[0::2], perm[1::2]])
    blk = 1280
    half_blocks = (p // 2) // blk
    ft = f_ij.T
    return pl.pallas_call(
        _wij_body,
        grid=(nblk,),
        in_specs=[
            pl.BlockSpec((r, blk), lambda i: (0, i + blk0)),
            pl.BlockSpec((r, blk), lambda i: (0, i + blk0 + half_blocks)),
            pl.BlockSpec((r, f), lambda i: (0, 0)),
            pl.BlockSpec((1, f), lambda i: (0, 0)),
        ],
        out_specs=pl.BlockSpec((blk, f), lambda i: (i, 0)),
        out_shape=jax.ShapeDtypeStruct((nblk * blk, f), jnp.int32),
    )(ft, ft, Wf.T[:, perm2], bf[perm2].reshape(1, -1))


def _out_body(p_ref, w2t_ref, b2_ref, o_ref):
    acc = p_ref[0] + p_ref[1]
    o_ref[...] = _ssp(jnp.dot(acc, w2t_ref[...],
                              preferred_element_type=jnp.float32) + b2_ref[...])


def _compute_out(parts, W2, b2):
    _, n, f = parts.shape
    d = W2.shape[0]
    blk = 1000
    grid = n // blk
    return pl.pallas_call(
        _out_body,
        grid=(grid,),
        in_specs=[
            pl.BlockSpec((2, blk, f), lambda i: (0, i, 0)),
            pl.BlockSpec((f, d), lambda i: (0, 0)),
            pl.BlockSpec((1, d), lambda i: (0, 0)),
        ],
        out_specs=pl.BlockSpec((blk, d), lambda i: (i, 0)),
        out_shape=jax.ShapeDtypeStruct((n, d), jnp.float32),
    )(parts, W2.T, b2.reshape(1, -1))


def _make_sc_scatter(n, d, p, chunk, row_start, tile_rows):
    nz_tiles = 10
    n_per_tile = n // nz_tiles
    half = chunk // 2
    p_half_tile = tile_rows
    n_chunks = p_half_tile // half
    mesh = plsc.VectorSubcoreMesh(core_axis_name="c", subcore_axis_name="s")

    @functools.partial(
        pl.kernel,
        out_type=jax.ShapeDtypeStruct((NC, n, d), jnp.float32),
        mesh=mesh,
        scratch_types=[
            pltpu.VMEM((2 * p_half_tile,), jnp.int32),
            pltpu.VMEM((2, half), jnp.int32),
            pltpu.VMEM((2, half), jnp.int32),
            pltpu.VMEM((2, half + LANES), jnp.float32),
            pltpu.VMEM((2, half + LANES), jnp.float32),
            pltpu.VMEM((chunk, d), jnp.float32),
            pltpu.VMEM((chunk, d), jnp.float32),
            pltpu.VMEM((half, d), jnp.int32),
            pltpu.VMEM((half, d), jnp.int32),
            pltpu.VMEM_SHARED((n, d), jnp.float32),
            pltpu.SemaphoreType.DMA,
            pltpu.SemaphoreType.DMA,
            pltpu.SemaphoreType.DMA,
            pltpu.SemaphoreType.DMA,
            pltpu.SemaphoreType.DMA,
            pltpu.SemaphoreType.DMA,
            pltpu.SemaphoreType.DMA,
            pltpu.SemaphoreType.DMA,
        ],
    )
    def sc_kernel(h_hbm, wij_hbm, idxj_hbm, idxi_hbm, rc_hbm, zero_hbm, out_hbm,
                  idxj_v, silo_v, sihi_v, rclo_v, rchi_v, rows0, rows1,
                  wij0, wij1, acc_sh,
                  gsem0, gsem1, wsem0, wsem1, ssem0, ssem1, isem0, isem1):
        rows = (rows0, rows1)
        wij = (wij0, wij1)
        gsem = (gsem0, gsem1)
        wsem = (wsem0, wsem1)
        ssem = (ssem0, ssem1)
        isem = (isem0, isem1)
        c = lax.axis_index("c")
        s = lax.axis_index("s")
        wid = c * NS + s

        row0 = s * n_per_tile

        @pl.when(s < nz_tiles)
        def _():
            pltpu.sync_copy(zero_hbm.at[c, pl.ds(row0, n_per_tile)],
                            acc_sh.at[pl.ds(row0, n_per_tile)])

        lo0 = row_start + wid * p_half_tile
        hi0 = (p // 2) + row_start + wid * p_half_tile
        wloc0 = wid * p_half_tile
        pltpu.sync_copy(idxj_hbm.at[pl.ds(lo0, p_half_tile)],
                        idxj_v.at[pl.ds(0, p_half_tile)])
        pltpu.sync_copy(idxj_hbm.at[pl.ds(hi0, p_half_tile)],
                        idxj_v.at[pl.ds(p_half_tile, p_half_tile)])
        plsc.subcore_barrier()

        def issue(j, b):
            off = j * half
            pltpu.async_copy(h_hbm.at[idxj_v.at[pl.ds(off, half)]],
                             rows[b].at[pl.ds(0, half)], gsem[b])
            pltpu.async_copy(h_hbm.at[idxj_v.at[pl.ds(p_half_tile + off, half)]],
                             rows[b].at[pl.ds(half, half)], gsem[b])
            pltpu.async_copy(wij_hbm.at[pl.ds(wloc0 + off, half)], wij[b],
                             wsem[b])
            pltpu.async_copy(idxi_hbm.at[pl.ds(lo0 + off, half)],
                             silo_v.at[b], isem[b])
            pltpu.async_copy(idxi_hbm.at[pl.ds(hi0 + off, half)],
                             sihi_v.at[b], isem[b])
            pltpu.async_copy(rc_hbm.at[pl.ds(lo0 + off, half)],
                             rclo_v.at[b, pl.ds(0, half)], isem[b])
            pltpu.async_copy(rc_hbm.at[pl.ds(hi0 + off, half)],
                             rchi_v.at[b, pl.ds(0, half)], isem[b])

        def drain_scatter(b):
            pltpu.make_async_copy(rows[b].at[pl.ds(0, half)],
                                  acc_sh.at[silo_v.at[b]], ssem[b]).wait()
            pltpu.make_async_copy(rows[b].at[pl.ds(half, half)],
                                  acc_sh.at[sihi_v.at[b]], ssem[b]).wait()

        def step(j, b):
            nb = 1 - b

            @pl.when(j + 1 < n_chunks)
            def _():
                @pl.when(j >= 1)
                def _():
                    drain_scatter(nb)
                issue(j + 1, nb)

            pltpu.make_async_copy(h_hbm.at[idxj_v.at[pl.ds(0, half)]],
                                  rows[b].at[pl.ds(0, half)], gsem[b]).wait()
            pltpu.make_async_copy(h_hbm.at[idxj_v.at[pl.ds(0, half)]],
                                  rows[b].at[pl.ds(half, half)], gsem[b]).wait()
            pltpu.make_async_copy(wij_hbm.at[pl.ds(0, half)], wij[b],
                                  wsem[b]).wait()
            for _ in range(4):
                pltpu.make_async_copy(rc_hbm.at[pl.ds(0, half)],
                                      rclo_v.at[b, pl.ds(0, half)],
                                      isem[b]).wait()
            zero16 = jnp.zeros((LANES, 1), jnp.int32)
            gdn = lax.GatherDimensionNumbers(offset_dims=(),
                                             collapsed_slice_dims=(0,),
                                             start_index_map=(0,))

            def mul_body(rr2, carry):
                for par, rowoff, rc_v in ((0, 0, rclo_v), (1, half, rchi_v)):
                    row = rowoff + rr2
                    rcv = rc_v[b, pl.ds(rr2, LANES)]
                    scale = lax.gather(
                        rcv, zero16, gdn, (1,),
                        mode=lax.GatherScatterMode.PROMISE_IN_BOUNDS)
                    for g in range(d // 32):
                        u = lax.bitcast_convert_type(
                            wij[b][rr2, pl.ds(64 * par + 16 * g, 16)],
                            jnp.uint32)
                        lo = lax.bitcast_convert_type(u << 16, jnp.float32) * scale
                        hi = lax.bitcast_convert_type(
                            u & jnp.uint32(0xFFFF0000), jnp.float32) * scale
                        sl0 = pl.ds(32 * g, LANES)
                        sl1 = pl.ds(32 * g + LANES, LANES)
                        rows[b][row, sl0] = rows[b][row, sl0] * lo
                        rows[b][row, sl1] = rows[b][row, sl1] * hi
                return carry

            lax.fori_loop(0, half, mul_body, 0)
            pltpu.async_copy(rows[b].at[pl.ds(0, half)],
                             acc_sh.at[silo_v.at[b]], ssem[b], add=True)
            pltpu.async_copy(rows[b].at[pl.ds(half, half)],
                             acc_sh.at[sihi_v.at[b]], ssem[b], add=True)

        issue(0, 0)

        def pair_body(t, carry):
            step(2 * t, 0)
            step(2 * t + 1, 1)
            return carry

        lax.fori_loop(0, n_chunks // 2, pair_body, 0)
        if n_chunks % 2 == 1:
            step(n_chunks - 1, 0)
        drain_scatter(1)
        drain_scatter(0)
        plsc.subcore_barrier()

        @pl.when(s < nz_tiles)
        def _():
            pltpu.sync_copy(acc_sh.at[pl.ds(row0, n_per_tile)],
                            out_hbm.at[c, pl.ds(row0, n_per_tile)])

    return sc_kernel


def kernel(x, f_ij, idx_i, idx_j, rcut_ij, W1, b1, Wf, bf, W2, b2):
    n, d = x.shape
    p = f_ij.shape[0]
    chunk = 80
    h = _compute_h(x, W1, b1)
    idxj = idx_j.astype(jnp.int32)
    idxi = idx_i.astype(jnp.int32)
    blk = 1280
    tile_rows_a, tile_rows_b = 1480, 3520
    nblk_a = tile_rows_a * NW // blk
    nblk_b = tile_rows_b * NW // blk
    wij_a = _compute_wij(f_ij, Wf, bf, 0, nblk_a)
    wij_b = _compute_wij(f_ij, Wf, bf, nblk_a, nblk_b)
    zeros = jnp.zeros((NC, n, d), jnp.float32)
    sc_a = _make_sc_scatter(n, d, p, chunk, 0, tile_rows_a)
    sc_b = _make_sc_scatter(n, d, p, chunk, tile_rows_a * NW, tile_rows_b)
    parts_a = sc_a(h, wij_a, idxj, idxi, rcut_ij, zeros)
    parts_b = sc_b(h, wij_b, idxj, idxi, rcut_ij, parts_a)
    return _compute_out(parts_b, W2, b2)
